# baseline (speedup 1.0000x reference)
import jax
import jax.numpy as jnp
from jax.experimental import pallas as pl
from jax.experimental.pallas import tpu as pltpu

_NBUF = 4
_BR = 512


def _gate_softmax_kernel(x_hbm, w_ref, o_ref, xbuf, sems):
    steps = x_hbm.shape[0] // _BR
    wb = w_ref[...].astype(jnp.bfloat16)

    def issue(step):
        slot = jax.lax.rem(step, _NBUF)
        pltpu.make_async_copy(
            x_hbm.at[pl.ds(step * _BR, _BR), :],
            xbuf.at[slot],
            sems.at[slot],
        ).start()

    for s in range(_NBUF - 1):
        issue(s)

    def body(i, carry):
        nxt = i + _NBUF - 1

        @pl.when(nxt < steps)
        def _():
            issue(nxt)

        slot = jax.lax.rem(i, _NBUF)
        pltpu.make_async_copy(
            x_hbm.at[pl.ds(i * _BR, _BR), :],
            xbuf.at[slot],
            sems.at[slot],
        ).wait()
        xb = xbuf[slot].astype(jnp.bfloat16)
        y = jax.lax.dot_general(
            xb, wb, (((1,), (1,)), ((), ())),
            preferred_element_type=jnp.float32,
        )
        m = jnp.max(y, axis=1, keepdims=True)
        e = jnp.exp(y - m)
        o_ref[pl.ds(i * _BR, _BR), :] = e / jnp.sum(e, axis=1, keepdims=True)
        return carry

    jax.lax.fori_loop(0, steps, body, 0)


def kernel(x, W):
    M, K = x.shape
    E = W.shape[0]
    return pl.pallas_call(
        _gate_softmax_kernel,
        in_specs=[
            pl.BlockSpec(memory_space=pl.ANY),
            pl.BlockSpec((E, K), lambda: (0, 0)),
        ],
        out_specs=pl.BlockSpec((M, E), lambda: (0, 0)),
        out_shape=jax.ShapeDtypeStruct((M, E), jnp.float32),
        scratch_shapes=[
            pltpu.VMEM((_NBUF, _BR, K), jnp.float32),
            pltpu.SemaphoreType.DMA((_NBUF,)),
        ],
    )(x, W)


# speedup vs baseline: 1.0022x; 1.0022x over previous
import jax
import jax.numpy as jnp
from jax.experimental import pallas as pl
from jax.experimental.pallas import tpu as pltpu

_NGRP = 4
_GRP = 4
_BR = 128
_GROWS = _GRP * _BR


def _gate_softmax_kernel(x_hbm, w_ref, o_ref, xbuf, sems):
    steps = x_hbm.shape[0] // _GROWS
    wb = w_ref[...].astype(jnp.bfloat16)

    def issue_group(g):
        grp = jax.lax.rem(g, _NGRP)
        for c in range(_GRP):
            pltpu.make_async_copy(
                x_hbm.at[pl.ds(g * _GROWS + c * _BR, _BR), :],
                xbuf.at[grp, c],
                sems.at[grp, c],
            ).start()

    def wait_group(g):
        grp = jax.lax.rem(g, _NGRP)
        for c in range(_GRP):
            pltpu.make_async_copy(
                x_hbm.at[pl.ds(g * _GROWS + c * _BR, _BR), :],
                xbuf.at[grp, c],
                sems.at[grp, c],
            ).wait()

    for g in range(_NGRP - 1):
        issue_group(g)

    def body(g, carry):
        nxt = g + _NGRP - 1

        @pl.when(nxt < steps)
        def _():
            issue_group(nxt)

        wait_group(g)
        grp = jax.lax.rem(g, _NGRP)
        xb = xbuf[grp].reshape(_GROWS, x_hbm.shape[1]).astype(jnp.bfloat16)
        y = jax.lax.dot_general(
            xb, wb, (((1,), (1,)), ((), ())),
            preferred_element_type=jnp.float32,
        )
        m = jnp.max(y, axis=1, keepdims=True)
        e = jnp.exp(y - m)
        o_ref[pl.ds(g * _GROWS, _GROWS), :] = e / jnp.sum(e, axis=1, keepdims=True)
        return carry

    jax.lax.fori_loop(0, steps, body, 0)


def kernel(x, W):
    M, K = x.shape
    E = W.shape[0]
    return pl.pallas_call(
        _gate_softmax_kernel,
        in_specs=[
            pl.BlockSpec(memory_space=pl.ANY),
            pl.BlockSpec((E, K), lambda: (0, 0)),
        ],
        out_specs=pl.BlockSpec((M, E), lambda: (0, 0)),
        out_shape=jax.ShapeDtypeStruct((M, E), jnp.float32),
        scratch_shapes=[
            pltpu.VMEM((_NGRP, _GRP, _BR, K), jnp.float32),
            pltpu.SemaphoreType.DMA((_NGRP, _GRP)),
        ],
    )(x, W)
